# baseline (speedup 1.0000x reference)
import jax
import jax.numpy as jnp
from jax.experimental import pallas as pl
from jax.experimental.pallas import tpu as pltpu

N = 10000
D = 128
TM = 320


def _fused_kernel(m_ref, adj_ref, x_ref, w_ref, b_ref, a_ref, out_ref,
                  vxp_ref, v_ref):
    @pl.when(pl.program_id(0) == 0)
    def _node_stage():
        xp = jnp.dot(x_ref[...], w_ref[...],
                     preferred_element_type=jnp.float32)
        h = jnp.tanh(xp + b_ref[...])
        e = jnp.dot(h, a_ref[...], preferred_element_type=jnp.float32)
        valid = jnp.sum(h, axis=1, keepdims=True) != 0.0
        c = jnp.sum(jnp.abs(a_ref[...]))
        v = jnp.where(valid, jnp.exp(e - c), 0.0)
        v_ref[...] = v
        vxp_ref[...] = v * xp

    a01 = (adj_ref[...] > 0.0).astype(jnp.float32)
    num = jax.lax.dot_general(
        a01, vxp_ref[...], (((1,), (0,)), ((), ())),
        preferred_element_type=jnp.float32)
    den = jax.lax.dot_general(
        a01, v_ref[...], (((1,), (0,)), ((), ())),
        preferred_element_type=jnp.float32)
    row = (pl.program_id(0) * TM
           + jax.lax.broadcasted_iota(jnp.int32, (TM, 1), 0))
    keep = (den > 0.0) & (row < m_ref[0])
    out_ref[...] = jnp.where(keep, num / den, 0.0)


def kernel(input, adj, M, W, b, a):
    x = input.astype(jnp.float32)
    b2 = b.reshape(1, D).astype(jnp.float32)
    m_arr = jnp.asarray(M, dtype=jnp.int32).reshape(1)
    out = pl.pallas_call(
        _fused_kernel,
        grid=(pl.cdiv(N, TM),),
        in_specs=[
            pl.BlockSpec(memory_space=pltpu.SMEM),
            pl.BlockSpec((TM, N), lambda i: (i, 0)),
            pl.BlockSpec((N, D), lambda i: (0, 0)),
            pl.BlockSpec((D, D), lambda i: (0, 0)),
            pl.BlockSpec((1, D), lambda i: (0, 0)),
            pl.BlockSpec((D, 1), lambda i: (0, 0)),
        ],
        out_specs=pl.BlockSpec((TM, D), lambda i: (i, 0)),
        out_shape=jax.ShapeDtypeStruct((N, D), jnp.float32),
        scratch_shapes=[
            pltpu.VMEM((N, D), jnp.float32),
            pltpu.VMEM((N, 1), jnp.float32),
        ],
    )(m_arr, adj, x, W.astype(jnp.float32), b2, a.astype(jnp.float32))
    return out


# speedup vs baseline: 1.0462x; 1.0462x over previous
import jax
import jax.numpy as jnp
from jax.experimental import pallas as pl
from jax.experimental.pallas import tpu as pltpu

N = 10000
D = 128
TM = 400


def _node_kernel(x_ref, w_ref, b_ref, a_ref, xp_ref, v_ref):
    xp = jnp.dot(x_ref[...], w_ref[...], preferred_element_type=jnp.float32)
    h = jnp.tanh(xp + b_ref[...])
    e = jnp.dot(h, a_ref[...], preferred_element_type=jnp.float32)
    valid = jnp.sum(h, axis=1, keepdims=True) != 0.0
    c = jnp.sum(jnp.abs(a_ref[...]))
    v = jnp.where(valid, jnp.exp(e - c), 0.0)
    xp_ref[...] = xp
    v_ref[...] = v


def _attn_kernel(m_ref, adj_ref, xp_ref, vr_ref, out_ref):
    aw = jnp.where(adj_ref[...] > 0.0, vr_ref[...], 0.0)
    num = jax.lax.dot_general(
        aw, xp_ref[...], (((1,), (0,)), ((), ())),
        preferred_element_type=jnp.float32)
    den = jnp.sum(aw, axis=1, keepdims=True)
    row = (pl.program_id(0) * TM
           + jax.lax.broadcasted_iota(jnp.int32, (TM, 1), 0))
    keep = (den > 0.0) & (row < m_ref[0])
    out_ref[...] = jnp.where(keep, num / den, 0.0)


def kernel(input, adj, M, W, b, a):
    x = input.astype(jnp.float32)
    b2 = b.reshape(1, D).astype(jnp.float32)
    xp, v = pl.pallas_call(
        _node_kernel,
        out_shape=(
            jax.ShapeDtypeStruct((N, D), jnp.float32),
            jax.ShapeDtypeStruct((N, 1), jnp.float32),
        ),
    )(x, W.astype(jnp.float32), b2, a.astype(jnp.float32))
    vr = v.reshape(1, N)
    m_arr = jnp.asarray(M, dtype=jnp.int32).reshape(1)
    out = pl.pallas_call(
        _attn_kernel,
        grid=(N // TM,),
        in_specs=[
            pl.BlockSpec(memory_space=pltpu.SMEM),
            pl.BlockSpec((TM, N), lambda i: (i, 0)),
            pl.BlockSpec((N, D), lambda i: (0, 0)),
            pl.BlockSpec((1, N), lambda i: (0, 0)),
        ],
        out_specs=pl.BlockSpec((TM, D), lambda i: (i, 0)),
        out_shape=jax.ShapeDtypeStruct((N, D), jnp.float32),
    )(m_arr, adj, xp, vr)
    return out
